# Initial kernel scaffold; baseline (speedup 1.0000x reference)
import jax
import jax.numpy as jnp
from jax.experimental import pallas as pl


def kernel(input_ids, emb):
    raise NotImplementedError("write your pallas kernel here")


# speedup vs baseline: 5.1737x; 5.1737x over previous
import dataclasses
import functools

import jax
import jax.numpy as jnp
from jax import lax
from jax.experimental import pallas as pl
from jax.experimental.pallas import tpu as pltpu
from jax.experimental.pallas import tpu_sc as plsc

VOCAB_SIZE = 32
DIM = 4
LANES = 16
CHUNK = 6400


def _body(total, ids_hbm, emb_hbm, out_hbm, table_vmem, sem):
    pltpu.async_copy(emb_hbm, table_vmem, sem).wait()
    lane_x4 = lax.iota(jnp.int32, LANES) * DIM

    def chunk_body(idx_vmem, out_vmem):
        @pl.loop(0, CHUNK, step=LANES)
        def _(g):
            iv = idx_vmem[pl.ds(g, LANES)] * DIM
            obase = g * DIM + lane_x4
            for d in range(DIM):
                vals = plsc.load_gather(table_vmem, [iv + d])
                plsc.store_scatter(out_vmem, [obase + d], vals)

    pltpu.emit_pipeline(
        chunk_body,
        grid=(total // CHUNK,),
        in_specs=[pl.BlockSpec((CHUNK,), lambda i: (i,))],
        out_specs=[pl.BlockSpec((CHUNK * DIM,), lambda i: (i,))],
        core_axis_name=("c", "s"),
        dimension_semantics=(pltpu.PARALLEL,),
    )(ids_hbm, out_hbm)


def kernel(input_ids, emb):
    n = input_ids.shape[0] * input_ids.shape[1]
    ids_flat = input_ids.reshape(-1).astype(jnp.int32)
    emb_flat = emb.reshape(-1)
    mesh = plsc.VectorSubcoreMesh(core_axis_name="c", subcore_axis_name="s")
    cp = dataclasses.replace(pltpu.CompilerParams(), needs_layout_passes=False)
    out = pl.kernel(
        functools.partial(_body, n),
        out_type=jax.ShapeDtypeStruct((n * DIM,), jnp.float32),
        mesh=mesh,
        scratch_types=[
            pltpu.VMEM((VOCAB_SIZE * DIM,), jnp.float32),
            pltpu.SemaphoreType.DMA,
        ],
        compiler_params=cp,
    )(ids_flat, emb_flat)
    return out.reshape(*input_ids.shape, DIM)


# speedup vs baseline: 65.5567x; 12.6712x over previous
import dataclasses
import functools

import jax
import jax.numpy as jnp
from jax import lax
from jax.experimental import pallas as pl
from jax.experimental.pallas import tpu as pltpu
from jax.experimental.pallas import tpu_sc as plsc

VOCAB_SIZE = 32
DIM = 4
LANES = 16


def _body(ids_hbm, emb_hbm, out_hbm, table_vmem, sem):
    pltpu.async_copy(emb_hbm, table_vmem, sem).wait()

    def block_body(idx_vmem, out_vmem):
        @pl.loop(0, 4)
        def _(ihl):
            @pl.loop(0, 8)
            def _(jl):
                @pl.loop(0, 128, step=LANES)
                def _(g):
                    iv = idx_vmem[0, ihl, jl, pl.ds(g, LANES)] * DIM
                    for d in range(DIM):
                        vals = plsc.load_gather(table_vmem, [iv + d])
                        out_vmem[jl, ihl, d, pl.ds(g, LANES)] = vals

    pltpu.emit_pipeline(
        block_body,
        grid=(25, 32),
        in_specs=[pl.BlockSpec((1, 4, 8, 128), lambda jt, itg: (jt, itg, 0, 0))],
        out_specs=[pl.BlockSpec((8, 4, 4, 128), lambda jt, itg: (jt, itg, 0, 0))],
        core_axis_name=("c", "s"),
        dimension_semantics=(pltpu.PARALLEL, pltpu.PARALLEL),
    )(ids_hbm, out_hbm)


def kernel(input_ids, emb):
    rows, cols = input_ids.shape
    assert (rows, cols) == (16384, 200) and emb.shape == (VOCAB_SIZE, DIM)
    ids_t = (
        input_ids.astype(jnp.int32)
        .reshape(rows // 128, 128, cols // 8, 8)
        .transpose(2, 0, 3, 1)
    )
    emb_flat = emb.reshape(-1)
    mesh = plsc.VectorSubcoreMesh(core_axis_name="c", subcore_axis_name="s")
    cp = dataclasses.replace(pltpu.CompilerParams(), needs_layout_passes=False)
    out = pl.kernel(
        _body,
        out_type=jax.ShapeDtypeStruct((cols, rows // 128, DIM, 128), jnp.float32),
        mesh=mesh,
        scratch_types=[
            pltpu.VMEM((VOCAB_SIZE * DIM,), jnp.float32),
            pltpu.SemaphoreType.DMA,
        ],
        compiler_params=cp,
    )(ids_t, emb_flat)
    return out.transpose(1, 3, 0, 2).reshape(rows, cols, DIM)
